# baseline (speedup 1.0000x reference)
import functools

import jax
import jax.numpy as jnp
from jax import lax
from jax.experimental import pallas as pl
from jax.experimental.pallas import tpu as pltpu
from jax.experimental.pallas import tpu_sc as plsc

_RES = 128
_C = 16
_V = _RES * _RES * _RES
_N = 500000

_NC = 2
_NS = 16
_NW = _NC * _NS
_L = 16

_CH = 128
_NCHUNK = -(-_N // _CH)
_KB = -(-_NCHUNK // _NW)
_TAIL_ROW = (_NCHUNK - 1) * _CH * _C // _RES
_TAIL_R = (_N * _C // _RES) - _TAIL_ROW
_NL = _NCHUNK * _CH
_OROWS = _N * _C // _RES

_TY = 8
_TVOX = _TY * _RES
_TPERZ = _RES // _TY
_TK = (_RES * _TPERZ) // _NW

_mesh = plsc.VectorSubcoreMesh(core_axis_name="c", subcore_axis_name="s")
_sc_params = pltpu.CompilerParams(use_tc_tiling_on_sc=False,
                                  needs_layout_passes=False)


@functools.partial(
    pl.kernel,
    mesh=_mesh,
    out_type=jax.ShapeDtypeStruct((_V, _C), jnp.float32),
    scratch_types=[
        pltpu.VMEM((2, _C, _TY, _RES), jnp.float32),
        pltpu.VMEM((2, _TVOX, _C + 1), jnp.float32),
        pltpu.SemaphoreType.DMA,
        pltpu.SemaphoreType.DMA,
        pltpu.SemaphoreType.DMA,
        pltpu.SemaphoreType.DMA,
    ],
    compiler_params=_sc_params,
)
def _sc_transpose(cube_hbm, table_hbm, in_v, out_v, si0, si1, so0, so1):
    wid = lax.axis_index("s") * _NC + lax.axis_index("c")
    lanes = jnp.arange(_L, dtype=jnp.int32)
    si = (si0, si1)
    so = (so0, so1)

    def in_desc(c, b):
        z = c // _TPERZ
        y0 = (c % _TPERZ) * _TY
        return pltpu.make_async_copy(cube_hbm.at[0, :, z, pl.ds(y0, _TY)],
                                     in_v.at[b], si[b])

    def out_desc(c, b):
        return pltpu.make_async_copy(
            out_v.at[b, :, pl.ds(0, _C)],
            table_hbm.at[pl.ds(c * _TVOX, _TVOX)], so[b])

    def compute(b):
        def yrow(yy, cy):
            def xgrp(xg, cx):
                rows = yy * _RES + xg * _L + lanes
                for f in range(_C):
                    vals = in_v[b, f, yy, pl.ds(xg * _L, _L)]
                    plsc.store_scatter(
                        out_v.at[b],
                        [rows, jnp.full((_L,), f, jnp.int32)], vals)
                return cx

            lax.fori_loop(0, _RES // _L, xgrp, None)
            return cy

        lax.fori_loop(0, _TY, yrow, None)

    for b in (0, 1):
        in_desc(wid * _TK + b, b).start()

    def pair(k2, carry):
        for b in (0, 1):
            k = k2 * 2 + b
            c = wid * _TK + k
            in_desc(c, b).wait()

            @pl.when(k2 > 0)
            def _():
                out_desc(c - 2, b).wait()

            compute(b)
            out_desc(c, b).start()

            @pl.when(k + 2 < _TK)
            def _():
                in_desc(c + 2, b).start()

        return carry

    lax.fori_loop(0, _TK // 2, pair, None)
    out_desc(wid * _TK + _TK - 2, 0).wait()
    out_desc(wid * _TK + _TK - 1, 1).wait()


_OFFS = [dz * _RES * _RES + dy * _RES + dx
         for dz in (0, 1) for dy in (0, 1) for dx in (0, 1)]


@functools.partial(
    pl.kernel,
    mesh=_mesh,
    out_type=jax.ShapeDtypeStruct((2, _NCHUNK, 8, _RES), jnp.float32),
    scratch_types=[
        pltpu.VMEM((2, 3, _CH), jnp.float32),
        pltpu.VMEM((2, 8, _CH), jnp.int32),
        pltpu.VMEM((2, 8, _CH), jnp.float32),
        pltpu.VMEM((2, 8, _CH, _L), jnp.float32),
        pltpu.VMEM((2, 2, 8, _CH + 5), jnp.float32),
        pltpu.SemaphoreType.DMA,
        pltpu.SemaphoreType.DMA,
        pltpu.SemaphoreType.DMA,
        pltpu.SemaphoreType.DMA,
        pltpu.SemaphoreType.DMA,
        pltpu.SemaphoreType.DMA,
    ],
    compiler_params=_sc_params,
)
def _sc_sample(table_hbm, lx_hbm, ly_hbm, lz_hbm, out_hbm,
               loc_v, idx_v, w_v, rows_v, out_v,
               sl0, sl1, sg0, sg1, so0, so1):
    wid = lax.axis_index("s") * _NC + lax.axis_index("c")
    sl = (sl0, sl1)
    sg = (sg0, sg1)
    so = (so0, so1)
    locs = (lx_hbm, ly_hbm, lz_hbm)

    def loc_descs(c, b):
        return [
            pltpu.make_async_copy(locs[a].at[pl.ds(c * _CH, _CH)],
                                  loc_v.at[b, a], sl[b])
            for a in range(3)
        ]

    def gather_descs(b):
        return [
            pltpu.make_async_copy(table_hbm.at[idx_v.at[b, j]],
                                  rows_v.at[b, j], sg[b])
            for j in range(8)
        ]

    def out_start_wait(c, b, start):
        d = pltpu.make_async_copy(
            out_v.at[b, :, :, pl.ds(0, _CH)], out_hbm.at[:, c], so[b])
        d.start() if start else d.wait()

    def axis_prep(b, a, g):
        i = jnp.clip(loc_v[b, a, pl.ds(g * _L, _L)] * 32.0 + 63.5,
                     0.0, 127.0)
        i0 = jnp.minimum(i.astype(jnp.int32), _RES - 2)
        f1 = i - i0.astype(jnp.float32)
        return i0, 1.0 - f1, f1

    def prep(b):
        def grp(g, cp):
            x0, wx0, wx1 = axis_prep(b, 0, g)
            y0, wy0, wy1 = axis_prep(b, 1, g)
            z0, wz0, wz1 = axis_prep(b, 2, g)
            vbase = (z0 * _RES + y0) * _RES + x0
            wz_ = (wz0, wz1)
            wy_ = (wy0, wy1)
            wx_ = (wx0, wx1)
            for j, off in enumerate(_OFFS):
                idx_v[b, j, pl.ds(g * _L, _L)] = vbase + off
                dz, dy, dx = j >> 2, (j >> 1) & 1, j & 1
                w_v[b, j, pl.ds(g * _L, _L)] = wz_[dz] * wy_[dy] * wx_[dx]
            return cp

        lax.fori_loop(0, _CH // _L, grp, None)

    lanes = jnp.arange(_L, dtype=jnp.int32)
    tvec = lanes // 8
    svec = lanes % 8

    def accum(b):
        def grp(g, cg):
            bb = g * _L
            wv = [w_v[b, j, pl.ds(bb, _L)] for j in range(8)]
            for l in range(_L):
                i = bb + l
                a0 = wv[0][l] * rows_v[b, 0, i] + wv[1][l] * rows_v[b, 1, i]
                a1 = wv[2][l] * rows_v[b, 2, i] + wv[3][l] * rows_v[b, 3, i]
                a2 = wv[4][l] * rows_v[b, 4, i] + wv[5][l] * rows_v[b, 5, i]
                a3 = wv[6][l] * rows_v[b, 6, i] + wv[7][l] * rows_v[b, 7, i]
                plsc.store_scatter(
                    out_v.at[b],
                    [tvec, svec, jnp.full((_L,), i, jnp.int32)],
                    (a0 + a1) + (a2 + a3))
            return cg

        lax.fori_loop(0, _CH // _L, grp, None)

    c0 = wid
    for cp in loc_descs(c0, 0):
        cp.start()
    for cp in loc_descs(c0, 0):
        cp.wait()
    prep(0)
    for cp in gather_descs(0):
        cp.start()
    for cp in loc_descs(_NW + wid, 1):
        cp.start()

    def pairbody(k2, carry):
        for b in (0, 1):
            k = k2 * 2 + b
            nb = 1 - b
            c = k * _NW + wid
            cn = c + _NW
            cnn = c + 2 * _NW

            @pl.when(cn < _NCHUNK)
            def _():
                for cp in loc_descs(cn, nb):
                    cp.wait()
                prep(nb)
                for cp in gather_descs(nb):
                    cp.start()

            @pl.when(cnn < _NCHUNK)
            def _():
                for cp in loc_descs(cnn, b):
                    cp.start()

            @pl.when(c < _NCHUNK)
            def _():
                for cp in gather_descs(b):
                    cp.wait()

                @pl.when(k >= 2)
                def _():
                    out_start_wait(c - 2 * _NW, b, False)

                accum(b)
                out_start_wait(c, b, True)

        return carry

    lax.fori_loop(0, (_KB + 1) // 2, pairbody, None)

    for kk in (_KB - 3, _KB - 2, _KB - 1):
        ct = kk * _NW + wid

        @pl.when((ct < _NCHUNK) & (ct + 2 * _NW >= _NCHUNK))
        def _():
            out_start_wait(ct, kk % 2, False)


def kernel(loc, cube):
    zpad = jnp.zeros((_NL - _N,), jnp.float32)
    lx = jnp.concatenate([loc[:, 0], zpad])
    ly = jnp.concatenate([loc[:, 1], zpad])
    lz = jnp.concatenate([loc[:, 2], zpad])
    table = _sc_transpose(cube)
    out = _sc_sample(table, lx, ly, lz)
    return out.transpose(1, 3, 0, 2).reshape(_NL, _C)[:_N]


# speedup vs baseline: 1.2570x; 1.2570x over previous
import functools

import jax
import jax.numpy as jnp
from jax import lax
from jax.experimental import pallas as pl
from jax.experimental.pallas import tpu as pltpu
from jax.experimental.pallas import tpu_sc as plsc

_RES = 128
_C = 16
_V = _RES * _RES * _RES
_N = 500000

_NC = 2
_NS = 16
_NW = _NC * _NS
_L = 16

_CH = 128
_NCHUNK = -(-_N // _CH)
_KB = -(-_NCHUNK // _NW)
_TAIL_ROW = (_NCHUNK - 1) * _CH * _C // _RES
_TAIL_R = (_N * _C // _RES) - _TAIL_ROW
_NL = _NCHUNK * _CH
_OROWS = _N * _C // _RES

_TY = 8
_TVOX = _TY * _RES
_TPERZ = _RES // _TY
_TK = (_RES * _TPERZ) // _NW

_mesh = plsc.VectorSubcoreMesh(core_axis_name="c", subcore_axis_name="s")
_sc_params = pltpu.CompilerParams(use_tc_tiling_on_sc=False,
                                  needs_layout_passes=False)


@functools.partial(
    pl.kernel,
    mesh=_mesh,
    out_type=jax.ShapeDtypeStruct((_V, _C), jnp.float32),
    scratch_types=[
        pltpu.VMEM((2, _C, _TY, _RES), jnp.float32),
        pltpu.VMEM((2, _TVOX, _C), jnp.float32),
        pltpu.SemaphoreType.DMA,
        pltpu.SemaphoreType.DMA,
        pltpu.SemaphoreType.DMA,
        pltpu.SemaphoreType.DMA,
    ],
    compiler_params=_sc_params,
)
def _sc_transpose(cube_hbm, table_hbm, in_v, out_v, si0, si1, so0, so1):
    wid = lax.axis_index("s") * _NC + lax.axis_index("c")
    lanes = jnp.arange(_L, dtype=jnp.int32)
    fdiag = [(lanes + d) % _C for d in range(_C)]
    si = (si0, si1)
    so = (so0, so1)

    def in_desc(c, b):
        z = c // _TPERZ
        y0 = (c % _TPERZ) * _TY
        return pltpu.make_async_copy(cube_hbm.at[0, :, z, pl.ds(y0, _TY)],
                                     in_v.at[b], si[b])

    def out_desc(c, b):
        return pltpu.make_async_copy(
            out_v.at[b], table_hbm.at[pl.ds(c * _TVOX, _TVOX)], so[b])

    def compute(b):
        def yrow(yy, cy):
            yvec = jnp.full((_L,), yy, jnp.int32)

            def xgrp(xg, cx):
                xvec = xg * _L + lanes
                rows = yy * _RES + xvec
                for d in range(_C):
                    vals = plsc.load_gather(in_v.at[b], [fdiag[d], yvec, xvec])
                    plsc.store_scatter(out_v.at[b], [rows, fdiag[d]], vals)
                return cx

            lax.fori_loop(0, _RES // _L, xgrp, None)
            return cy

        lax.fori_loop(0, _TY, yrow, None)

    for b in (0, 1):
        in_desc(wid * _TK + b, b).start()

    def pair(k2, carry):
        for b in (0, 1):
            k = k2 * 2 + b
            c = wid * _TK + k
            in_desc(c, b).wait()

            @pl.when(k2 > 0)
            def _():
                out_desc(c - 2, b).wait()

            compute(b)
            out_desc(c, b).start()

            @pl.when(k + 2 < _TK)
            def _():
                in_desc(c + 2, b).start()

        return carry

    lax.fori_loop(0, _TK // 2, pair, None)
    out_desc(wid * _TK + _TK - 2, 0).wait()
    out_desc(wid * _TK + _TK - 1, 1).wait()


_OFFS = [dz * _RES * _RES + dy * _RES + dx
         for dz in (0, 1) for dy in (0, 1) for dx in (0, 1)]


@functools.partial(
    pl.kernel,
    mesh=_mesh,
    out_type=jax.ShapeDtypeStruct((2, _NCHUNK, 8 * _RES), jnp.float32),
    scratch_types=[
        pltpu.VMEM((2, 3, _CH), jnp.float32),
        pltpu.VMEM((2, 8, _CH), jnp.int32),
        pltpu.VMEM((2, 8, _CH), jnp.float32),
        pltpu.VMEM((2, 8, _CH, _L), jnp.float32),
        pltpu.VMEM((2, 2, 8 * _RES), jnp.float32),
        pltpu.VMEM((_L * 18,), jnp.float32),
        pltpu.SemaphoreType.DMA,
        pltpu.SemaphoreType.DMA,
        pltpu.SemaphoreType.DMA,
        pltpu.SemaphoreType.DMA,
        pltpu.SemaphoreType.DMA,
        pltpu.SemaphoreType.DMA,
    ],
    compiler_params=_sc_params,
)
def _sc_sample(table_hbm, lx_hbm, ly_hbm, lz_hbm, out_hbm,
               loc_v, idx_v, w_v, rows_v, out_v, tile_v,
               sl0, sl1, sg0, sg1, so0, so1):
    wid = lax.axis_index("s") * _NC + lax.axis_index("c")
    sl = (sl0, sl1)
    sg = (sg0, sg1)
    so = (so0, so1)
    locs = (lx_hbm, ly_hbm, lz_hbm)

    def loc_descs(c, b):
        return [
            pltpu.make_async_copy(locs[a].at[pl.ds(c * _CH, _CH)],
                                  loc_v.at[b, a], sl[b])
            for a in range(3)
        ]

    def gather_descs(b):
        return [
            pltpu.make_async_copy(table_hbm.at[idx_v.at[b, j]],
                                  rows_v.at[b, j], sg[b])
            for j in range(8)
        ]

    def out_start_wait(c, b, start):
        d = pltpu.make_async_copy(out_v.at[b], out_hbm.at[:, c], so[b])
        d.start() if start else d.wait()

    def axis_prep(b, a, g):
        i = jnp.clip(loc_v[b, a, pl.ds(g * _L, _L)] * 32.0 + 63.5,
                     0.0, 127.0)
        i0 = jnp.minimum(i.astype(jnp.int32), _RES - 2)
        f1 = i - i0.astype(jnp.float32)
        return i0, 1.0 - f1, f1

    def prep(b):
        def grp(g, cp):
            x0, wx0, wx1 = axis_prep(b, 0, g)
            y0, wy0, wy1 = axis_prep(b, 1, g)
            z0, wz0, wz1 = axis_prep(b, 2, g)
            vbase = (z0 * _RES + y0) * _RES + x0
            wz_ = (wz0, wz1)
            wy_ = (wy0, wy1)
            wx_ = (wx0, wx1)
            for j, off in enumerate(_OFFS):
                idx_v[b, j, pl.ds(g * _L, _L)] = vbase + off
                dz, dy, dx = j >> 2, (j >> 1) & 1, j & 1
                w_v[b, j, pl.ds(g * _L, _L)] = wz_[dz] * wy_[dy] * wx_[dx]
            return cp

        lax.fori_loop(0, _CH // _L, grp, None)

    lanes = jnp.arange(_L, dtype=jnp.int32)
    diag_idx = [lanes * 18 + (lanes + d) % _C for d in range(_C)]
    out_off = [((lanes + d) % _C // 8) * (8 * _RES)
               + ((lanes + d) % _C % 8) * _RES for d in range(_C)]

    def accum(b):
        def grp(g, cg):
            bb = g * _L
            wv = [w_v[b, j, pl.ds(bb, _L)] for j in range(8)]
            for l in range(_L):
                i = bb + l
                a0 = wv[0][l] * rows_v[b, 0, i] + wv[1][l] * rows_v[b, 1, i]
                a1 = wv[2][l] * rows_v[b, 2, i] + wv[3][l] * rows_v[b, 3, i]
                a2 = wv[4][l] * rows_v[b, 4, i] + wv[5][l] * rows_v[b, 5, i]
                a3 = wv[6][l] * rows_v[b, 6, i] + wv[7][l] * rows_v[b, 7, i]
                tile_v[pl.ds(l * 18, _L)] = (a0 + a1) + (a2 + a3)
            ivec = bb + lanes
            for d in range(_C):
                vals = plsc.load_gather(tile_v, [diag_idx[d]])
                plsc.store_scatter(
                    out_v.at[b],
                    [out_off[d] // (8 * _RES), out_off[d] % (8 * _RES) + ivec],
                    vals)
            return cg

        lax.fori_loop(0, _CH // _L, grp, None)

    c0 = wid
    for cp in loc_descs(c0, 0):
        cp.start()
    for cp in loc_descs(c0, 0):
        cp.wait()
    prep(0)
    for cp in gather_descs(0):
        cp.start()
    for cp in loc_descs(_NW + wid, 1):
        cp.start()

    def pairbody(k2, carry):
        for b in (0, 1):
            k = k2 * 2 + b
            nb = 1 - b
            c = k * _NW + wid
            cn = c + _NW
            cnn = c + 2 * _NW

            @pl.when(cn < _NCHUNK)
            def _():
                for cp in loc_descs(cn, nb):
                    cp.wait()
                prep(nb)
                for cp in gather_descs(nb):
                    cp.start()

            @pl.when(cnn < _NCHUNK)
            def _():
                for cp in loc_descs(cnn, b):
                    cp.start()

            @pl.when(c < _NCHUNK)
            def _():
                for cp in gather_descs(b):
                    cp.wait()

                @pl.when(k >= 2)
                def _():
                    out_start_wait(c - 2 * _NW, b, False)

                accum(b)
                out_start_wait(c, b, True)

        return carry

    lax.fori_loop(0, (_KB + 1) // 2, pairbody, None)

    for kk in (_KB - 3, _KB - 2, _KB - 1):
        ct = kk * _NW + wid

        @pl.when((ct < _NCHUNK) & (ct + 2 * _NW >= _NCHUNK))
        def _():
            out_start_wait(ct, kk % 2, False)


def kernel(loc, cube):
    zpad = jnp.zeros((_NL - _N,), jnp.float32)
    lx = jnp.concatenate([loc[:, 0], zpad])
    ly = jnp.concatenate([loc[:, 1], zpad])
    lz = jnp.concatenate([loc[:, 2], zpad])
    table = _sc_transpose(cube)
    out = _sc_sample(table, lx, ly, lz)
    return (out.reshape(2, _NCHUNK, 8, _RES)
            .transpose(1, 3, 0, 2).reshape(_NL, _C)[:_N])
